# Initial kernel scaffold; baseline (speedup 1.0000x reference)
import jax
import jax.numpy as jnp
from jax.experimental import pallas as pl


def kernel(disp, foregroundMask, centerx_raw, centery_raw, bx_raw, by_raw):
    raise NotImplementedError("write your pallas kernel here")


# speedup vs baseline: 3.9108x; 3.9108x over previous
import functools

import jax
import jax.numpy as jnp
from jax import lax
from jax.experimental import pallas as pl
from jax.experimental.pallas import tpu as pltpu
from jax.experimental.pallas import tpu_sc as plsc

WD = 11
PTS = 5000
DENSE = 20
BN = 10
H = 512
W = 1024
HW = H * W
P0 = PTS * BN

NC = 2
NS = 16
NWK = NC * NS
CHUNK = 128
NCHUNK = 13
PW = CHUNK * NCHUNK
PTOT = NWK * PW
GRP = CHUNK // 16


def _roll(x, shift, axis):
    return pltpu.roll(x, shift % x.shape[axis], axis)


def _pack_body(mask_ref, disp_ref, out_ref):
    m = mask_ref[0]
    d = disp_ref[0]
    cs = _roll(m, 1, 0) + 2.0 * m + _roll(m, -1, 0)
    g = jnp.abs(_roll(cs, -1, 1) - _roll(cs, 1, 1))
    rm = jnp.maximum(jnp.maximum(_roll(g, -1, 1), g), _roll(g, 1, 1))
    pm = jnp.maximum(jnp.maximum(_roll(rm, -1, 0), rm), _roll(rm, 1, 0))
    border = (pm > 3.1) & (d > 0.007)
    bits = lax.bitcast_convert_type(d, jnp.int32)
    bits = bits | jnp.where(m > 0.5, jnp.int32(-(2 ** 31)), jnp.int32(0))
    bits = bits | jnp.where(border, jnp.int32(1 << 30), jnp.int32(0))
    out_ref[0] = bits


def _pack_image(mask3, disp3):
    return pl.pallas_call(
        _pack_body,
        grid=(BN,),
        in_specs=[pl.BlockSpec((1, H, W), lambda b: (b, 0, 0)),
                  pl.BlockSpec((1, H, W), lambda b: (b, 0, 0))],
        out_specs=pl.BlockSpec((1, H, W), lambda b: (b, 0, 0)),
        out_shape=jax.ShapeDtypeStruct((BN, H, W), jnp.int32),
    )(mask3, disp3)


def _sqrt16(x):
    i = lax.bitcast_convert_type(x, jnp.int32)
    y = lax.bitcast_convert_type(jnp.int32(0x1FBD1DF5) + (i >> 1), jnp.float32)
    for _ in range(4):
        y = 0.5 * (y + x / y)
    return y


def _sc_body(packed_hbm, cx_hbm, cy_hbm, bxf_hbm, byf_hbm, out_hbm,
             cxv, cyv, bxv, byv, addrv, datav, caddrv, cdatav, accv, sem):
    wid = lax.axis_index("s") * NC + lax.axis_index("c")
    zero16 = jnp.zeros((16,), jnp.float32)
    for i in range(5):
        accv[i] = zero16

    def chunk_body(c, _):
        base = pl.multiple_of(wid * PW + c * CHUNK, CHUNK)
        pltpu.sync_copy(cx_hbm.at[pl.ds(base, CHUNK)], cxv)
        pltpu.sync_copy(cy_hbm.at[pl.ds(base, CHUNK)], cyv)
        sbase = pl.multiple_of(base * DENSE, CHUNK)
        pltpu.sync_copy(bxf_hbm.at[pl.ds(sbase, CHUNK * DENSE)], bxv)
        pltpu.sync_copy(byf_hbm.at[pl.ds(sbase, CHUNK * DENSE)], byv)

        def addr_body(g, _):
            off = pl.multiple_of(g * 16, 16)
            cxg = cxv[pl.ds(off, 16)] + WD
            cyg = cyv[pl.ds(off, 16)] + WD
            pvec = base + off + lax.iota(jnp.int32, 16)
            ch = jnp.minimum(pvec // PTS, BN - 1)
            ac = ch * HW + cyg * W + cxg
            caddrv[pl.ds(off, 16)] = ac
            bidx = (off + lax.iota(jnp.int32, 16)) * DENSE
            for j in range(DENSE):
                bx = plsc.load_gather(bxv, [bidx + j]) - WD
                by = plsc.load_gather(byv, [bidx + j]) - 7
                a = ac + by * W + bx
                a = jnp.clip(a, 0, BN * HW - 1)
                addrv[j, pl.ds(off, 16)] = a
            return 0

        lax.fori_loop(0, GRP, addr_body, 0)

        copies = [pltpu.async_copy(packed_hbm.at[addrv.at[j]], datav.at[j], sem)
                  for j in range(DENSE)]
        copies.append(pltpu.async_copy(packed_hbm.at[caddrv], cdatav, sem))
        for cp in copies:
            cp.wait()

        def stats_body(g, _):
            off = pl.multiple_of(g * 16, 16)
            cword = cdatav[pl.ds(off, 16)]
            onb = jnp.where((cword & (1 << 30)) != 0, 1.0, 0.0).astype(jnp.float32)
            pvec = base + off + lax.iota(jnp.int32, 16)
            onb = onb * jnp.where(pvec < P0, 1.0, 0.0).astype(jnp.float32)

            posN = zero16
            sumP = zero16
            sumA = zero16
            for j in range(DENSE):
                w = datav[j, pl.ds(off, 16)]
                t = jnp.where(w < 0, 1.0, 0.0).astype(jnp.float32)
                dv = lax.bitcast_convert_type(w & jnp.int32(0x3FFFFFFF),
                                              jnp.float32)
                posN = posN + t
                sumP = sumP + dv * t
                sumA = sumA + dv
            negN = jnp.float32(DENSE) - posN
            posDen = jnp.maximum(posN, 1.0)
            negDen = jnp.maximum(negN, 1.0)
            posMean = sumP / posDen
            negMean = (sumA - sumP) / negDen
            vP = zero16
            vN = zero16
            for j in range(DENSE):
                w = datav[j, pl.ds(off, 16)]
                t = jnp.where(w < 0, 1.0, 0.0).astype(jnp.float32)
                dv = lax.bitcast_convert_type(w & jnp.int32(0x3FFFFFFF),
                                              jnp.float32)
                ep = dv - posMean
                en = dv - negMean
                vP = vP + ep * ep * t
                vN = vN + en * en * (1.0 - t)
            simP = _sqrt16(vP / posDen + 1e-14)
            simN = _sqrt16(vN / negDen + 1e-14)
            balance = jnp.where((posN > 4.5) & (negN > 4.5), 1.0, 0.0)
            sel = onb * balance.astype(jnp.float32)
            plsc.addupdate(accv.at[0], sel)
            plsc.addupdate(accv.at[1], simP * sel)
            plsc.addupdate(accv.at[2], simN * sel)
            plsc.addupdate(accv.at[3], (negMean - posMean) * sel)
            plsc.addupdate(accv.at[4], onb)
            return 0

        lax.fori_loop(0, GRP, stats_body, 0)
        return 0

    lax.fori_loop(0, NCHUNK, chunk_body, 0)
    pltpu.sync_copy(accv, out_hbm.at[wid])


def _sc_sample(packed_flat, cxp, cyp, bxp, byp):
    mesh = plsc.VectorSubcoreMesh(core_axis_name="c", subcore_axis_name="s")
    return pl.kernel(
        _sc_body,
        out_type=jax.ShapeDtypeStruct((NWK, 5, 16), jnp.float32),
        mesh=mesh,
        compiler_params=pltpu.CompilerParams(needs_layout_passes=False),
        scratch_types=[
            pltpu.VMEM((CHUNK,), jnp.int32),
            pltpu.VMEM((CHUNK,), jnp.int32),
            pltpu.VMEM((CHUNK * DENSE,), jnp.int32),
            pltpu.VMEM((CHUNK * DENSE,), jnp.int32),
            pltpu.VMEM((DENSE, CHUNK), jnp.int32),
            pltpu.VMEM((DENSE, CHUNK), jnp.int32),
            pltpu.VMEM((CHUNK,), jnp.int32),
            pltpu.VMEM((CHUNK,), jnp.int32),
            pltpu.VMEM((5, 16), jnp.float32),
            pltpu.SemaphoreType.DMA,
        ],
    )(packed_flat, cxp, cyp, bxp, byp)


def kernel(disp, foregroundMask, centerx_raw, centery_raw, bx_raw, by_raw):
    mask3 = foregroundMask.reshape(BN, H, W)
    disp3 = disp.reshape(BN, H, W)
    packed = _pack_image(mask3, disp3).reshape(BN * HW)

    cxp = jnp.zeros((PTOT,), jnp.int32).at[:P0].set(centerx_raw.astype(jnp.int32))
    cyp = jnp.zeros((PTOT,), jnp.int32).at[:P0].set(centery_raw.astype(jnp.int32))
    bxp = jnp.zeros((PTOT * DENSE,), jnp.int32).at[:P0 * DENSE].set(
        bx_raw.astype(jnp.int32).reshape(-1))
    byp = jnp.zeros((PTOT * DENSE,), jnp.int32).at[:P0 * DENSE].set(
        by_raw.astype(jnp.int32).reshape(-1))

    parts = _sc_sample(packed, cxp, cyp, bxp, byp).sum(axis=(0, 2))
    count = parts[0]
    countSafe = jnp.where(count > 0, count, jnp.float32(1.0))
    lossSim = (parts[1] + parts[2]) / countSafe * jnp.float32(0.5)
    lossContrast = parts[3] / countSafe + jnp.float32(0.02)
    valid = (parts[4] >= 100) & (count >= 100)
    return (jnp.where(valid, lossSim, jnp.float32(0.0)),
            jnp.where(valid, lossContrast, jnp.float32(0.0)))


# speedup vs baseline: 8.1457x; 2.0829x over previous
import functools

import jax
import jax.numpy as jnp
from jax import lax
from jax.experimental import pallas as pl
from jax.experimental.pallas import tpu as pltpu
from jax.experimental.pallas import tpu_sc as plsc

WD = 11
PTS = 5000
DENSE = 20
BN = 10
H = 512
W = 1024
HW = H * W
P0 = PTS * BN

NC = 2
NS = 16
NWK = NC * NS
CHUNK = 128
NCHUNK = 13
PW = CHUNK * NCHUNK
PTOT = NWK * PW
GRP = CHUNK // 16


def _roll(x, shift, axis):
    return pltpu.roll(x, shift % x.shape[axis], axis)


def _pack_body(mask_ref, disp_ref, out_ref):
    m = mask_ref[0]
    d = disp_ref[0]
    cs = _roll(m, 1, 0) + 2.0 * m + _roll(m, -1, 0)
    g = jnp.abs(_roll(cs, -1, 1) - _roll(cs, 1, 1))
    rm = jnp.maximum(jnp.maximum(_roll(g, -1, 1), g), _roll(g, 1, 1))
    pm = jnp.maximum(jnp.maximum(_roll(rm, -1, 0), rm), _roll(rm, 1, 0))
    border = (pm > 3.1) & (d > 0.007)
    bits = lax.bitcast_convert_type(d, jnp.int32)
    bits = bits | jnp.where(m > 0.5, jnp.int32(-(2 ** 31)), jnp.int32(0))
    bits = bits | jnp.where(border, jnp.int32(1 << 30), jnp.int32(0))
    out_ref[0] = bits


def _pack_image(mask3, disp3):
    return pl.pallas_call(
        _pack_body,
        grid=(BN,),
        in_specs=[pl.BlockSpec((1, H, W), lambda b: (b, 0, 0)),
                  pl.BlockSpec((1, H, W), lambda b: (b, 0, 0))],
        out_specs=pl.BlockSpec((1, H, W), lambda b: (b, 0, 0)),
        out_shape=jax.ShapeDtypeStruct((BN, H, W), jnp.int32),
    )(mask3, disp3)


def _sqrt16(x):
    i = lax.bitcast_convert_type(x, jnp.int32)
    y = lax.bitcast_convert_type(jnp.int32(0x1FBD1DF5) + (i >> 1), jnp.float32)
    for _ in range(4):
        y = 0.5 * (y + x / y)
    return y


def _sc_body(packed_hbm, cx_hbm, cy_hbm, bxf_hbm, byf_hbm, out_hbm,
             cxv, cyv, bxv, byv, addrv, datav, caddrv, cdatav, accv, sem):
    wid = lax.axis_index("s") * NC + lax.axis_index("c")
    zero16 = jnp.zeros((16,), jnp.float32)
    for i in range(5):
        accv[i] = zero16

    def chunk_body(c, _):
        base = wid * PW + c * CHUNK
        bc = pl.multiple_of(jnp.minimum(base, P0 - CHUNK), 16)
        cps = [pltpu.async_copy(cx_hbm.at[pl.ds(bc, CHUNK)], cxv, sem),
               pltpu.async_copy(cy_hbm.at[pl.ds(bc, CHUNK)], cyv, sem),
               pltpu.async_copy(
                   bxf_hbm.at[pl.ds(pl.multiple_of(bc * DENSE, 16),
                                    CHUNK * DENSE)], bxv, sem),
               pltpu.async_copy(
                   byf_hbm.at[pl.ds(pl.multiple_of(bc * DENSE, 16),
                                    CHUNK * DENSE)], byv, sem)]
        for cp in cps:
            cp.wait()

        def addr_body(g, _):
            off = pl.multiple_of(g * 16, 16)
            cxg = cxv[pl.ds(off, 16)] + WD
            cyg = cyv[pl.ds(off, 16)] + WD
            pvec = bc + off + lax.iota(jnp.int32, 16)
            ch = (pvec.astype(jnp.float32) * jnp.float32(1.0 / PTS)
                  ).astype(jnp.int32)
            ac = ch * HW + cyg * W + cxg
            caddrv[pl.ds(off, 16)] = ac
            bidx = (off + lax.iota(jnp.int32, 16)) * DENSE
            for j in range(DENSE):
                bx = plsc.load_gather(bxv, [bidx + j]) - WD
                by = plsc.load_gather(byv, [bidx + j]) - 7
                addrv[j, pl.ds(off, 16)] = ac + by * W + bx
            return 0

        lax.fori_loop(0, GRP, addr_body, 0)

        copies = [pltpu.async_copy(packed_hbm.at[addrv.at[j]], datav.at[j], sem)
                  for j in range(DENSE)]
        copies.append(pltpu.async_copy(packed_hbm.at[caddrv], cdatav, sem))
        for cp in copies:
            cp.wait()

        def stats_body(g, _):
            off = pl.multiple_of(g * 16, 16)
            cword = cdatav[pl.ds(off, 16)]
            onb = jnp.where((cword & (1 << 30)) != 0, 1.0, 0.0).astype(jnp.float32)
            pvec = bc + off + lax.iota(jnp.int32, 16)
            live = (pvec >= base) & (pvec < P0)
            onb = onb * jnp.where(live, 1.0, 0.0).astype(jnp.float32)

            posN = zero16
            sumP = zero16
            sumA = zero16
            for j in range(DENSE):
                w = datav[j, pl.ds(off, 16)]
                t = jnp.where(w < 0, 1.0, 0.0).astype(jnp.float32)
                dv = lax.bitcast_convert_type(w & jnp.int32(0x3FFFFFFF),
                                              jnp.float32)
                posN = posN + t
                sumP = sumP + dv * t
                sumA = sumA + dv
            negN = jnp.float32(DENSE) - posN
            posDen = jnp.maximum(posN, 1.0)
            negDen = jnp.maximum(negN, 1.0)
            posMean = sumP / posDen
            negMean = (sumA - sumP) / negDen
            vP = zero16
            vN = zero16
            for j in range(DENSE):
                w = datav[j, pl.ds(off, 16)]
                t = jnp.where(w < 0, 1.0, 0.0).astype(jnp.float32)
                dv = lax.bitcast_convert_type(w & jnp.int32(0x3FFFFFFF),
                                              jnp.float32)
                ep = dv - posMean
                en = dv - negMean
                vP = vP + ep * ep * t
                vN = vN + en * en * (1.0 - t)
            simP = _sqrt16(vP / posDen + 1e-14)
            simN = _sqrt16(vN / negDen + 1e-14)
            balance = jnp.where((posN > 4.5) & (negN > 4.5), 1.0, 0.0)
            sel = onb * balance.astype(jnp.float32)
            plsc.addupdate(accv.at[0], sel)
            plsc.addupdate(accv.at[1], simP * sel)
            plsc.addupdate(accv.at[2], simN * sel)
            plsc.addupdate(accv.at[3], (negMean - posMean) * sel)
            plsc.addupdate(accv.at[4], onb)
            return 0

        lax.fori_loop(0, GRP, stats_body, 0)
        return 0

    lax.fori_loop(0, NCHUNK, chunk_body, 0)
    pltpu.sync_copy(accv, out_hbm.at[wid])


def _sc_sample(packed_flat, cxp, cyp, bxp, byp):
    mesh = plsc.VectorSubcoreMesh(core_axis_name="c", subcore_axis_name="s")
    return pl.kernel(
        _sc_body,
        out_type=jax.ShapeDtypeStruct((NWK, 5, 16), jnp.float32),
        mesh=mesh,
        compiler_params=pltpu.CompilerParams(needs_layout_passes=False),
        scratch_types=[
            pltpu.VMEM((CHUNK,), jnp.int32),
            pltpu.VMEM((CHUNK,), jnp.int32),
            pltpu.VMEM((CHUNK * DENSE,), jnp.int32),
            pltpu.VMEM((CHUNK * DENSE,), jnp.int32),
            pltpu.VMEM((DENSE, CHUNK), jnp.int32),
            pltpu.VMEM((DENSE, CHUNK), jnp.int32),
            pltpu.VMEM((CHUNK,), jnp.int32),
            pltpu.VMEM((CHUNK,), jnp.int32),
            pltpu.VMEM((5, 16), jnp.float32),
            pltpu.SemaphoreType.DMA,
        ],
    )(packed_flat, cxp, cyp, bxp, byp)


def kernel(disp, foregroundMask, centerx_raw, centery_raw, bx_raw, by_raw):
    mask3 = foregroundMask.reshape(BN, H, W)
    disp3 = disp.reshape(BN, H, W)
    packed = _pack_image(mask3, disp3).reshape(BN * HW)

    cxp = centerx_raw.astype(jnp.int32)
    cyp = centery_raw.astype(jnp.int32)
    bxp = bx_raw.astype(jnp.int32).reshape(-1)
    byp = by_raw.astype(jnp.int32).reshape(-1)

    parts = _sc_sample(packed, cxp, cyp, bxp, byp).sum(axis=(0, 2))
    count = parts[0]
    countSafe = jnp.where(count > 0, count, jnp.float32(1.0))
    lossSim = (parts[1] + parts[2]) / countSafe * jnp.float32(0.5)
    lossContrast = parts[3] / countSafe + jnp.float32(0.02)
    valid = (parts[4] >= 100) & (count >= 100)
    return (jnp.where(valid, lossSim, jnp.float32(0.0)),
            jnp.where(valid, lossContrast, jnp.float32(0.0)))


# speedup vs baseline: 9.2725x; 1.1383x over previous
import functools

import jax
import jax.numpy as jnp
from jax import lax
from jax.experimental import pallas as pl
from jax.experimental.pallas import tpu as pltpu
from jax.experimental.pallas import tpu_sc as plsc

WD = 11
PTS = 5000
DENSE = 20
BN = 10
H = 512
W = 1024
HW = H * W
P0 = PTS * BN

NC = 2
NS = 16
NWK = NC * NS
CHUNK = 128
NCHUNK = 13
PW = CHUNK * NCHUNK
PTOT = NWK * PW
GRP = CHUNK // 16


def _roll(x, shift, axis):
    return pltpu.roll(x, shift % x.shape[axis], axis)


def _pack_body(mask_ref, disp_ref, out_ref):
    m = mask_ref[0]
    d = disp_ref[0]
    cs = _roll(m, 1, 0) + 2.0 * m + _roll(m, -1, 0)
    g = jnp.abs(_roll(cs, -1, 1) - _roll(cs, 1, 1))
    rm = jnp.maximum(jnp.maximum(_roll(g, -1, 1), g), _roll(g, 1, 1))
    pm = jnp.maximum(jnp.maximum(_roll(rm, -1, 0), rm), _roll(rm, 1, 0))
    border = (pm > 3.1) & (d > 0.007)
    bits = lax.bitcast_convert_type(d, jnp.int32)
    bits = bits | jnp.where(m > 0.5, jnp.int32(-(2 ** 31)), jnp.int32(0))
    bits = bits | jnp.where(border, jnp.int32(1 << 30), jnp.int32(0))
    out_ref[0] = bits


def _pack_image(mask3, disp3):
    return pl.pallas_call(
        _pack_body,
        grid=(BN,),
        in_specs=[pl.BlockSpec((1, H, W), lambda b: (b, 0, 0)),
                  pl.BlockSpec((1, H, W), lambda b: (b, 0, 0))],
        out_specs=pl.BlockSpec((1, H, W), lambda b: (b, 0, 0)),
        out_shape=jax.ShapeDtypeStruct((BN, H, W), jnp.int32),
    )(mask3, disp3)


def _sqrt16(x):
    i = lax.bitcast_convert_type(x, jnp.int32)
    y = lax.bitcast_convert_type(jnp.int32(0x1FBD1DF5) + (i >> 1), jnp.float32)
    for _ in range(4):
        y = 0.5 * (y + x / y)
    return y


def _sc_body(packed_hbm, cx_hbm, cy_hbm, bxf_hbm, byf_hbm, out_hbm,
             cxv0, cxv1, cyv0, cyv1, bxv0, bxv1, byv0, byv1,
             addrv0, addrv1, datav0, datav1, caddrv0, caddrv1,
             cdatav0, cdatav1, accv, sem_s0, sem_s1, sem_g0, sem_g1):
    wid = lax.axis_index("s") * NC + lax.axis_index("c")
    zero16 = jnp.zeros((16,), jnp.float32)
    for i in range(5):
        accv[i] = zero16
    cxv, cyv, bxv, byv = (cxv0, cxv1), (cyv0, cyv1), (bxv0, bxv1), (byv0, byv1)
    addrv, datav = (addrv0, addrv1), (datav0, datav1)
    caddrv, cdatav = (caddrv0, caddrv1), (cdatav0, cdatav1)
    sem_s = (sem_s0, sem_s1)
    sem_g = (sem_g0, sem_g1)

    def bases(c):
        base = wid * PW + c * CHUNK
        bc = pl.multiple_of(jnp.minimum(base, P0 - CHUNK), 16)
        return base, bc

    def fire_stage(c, b):
        _, bc = bases(c)
        sb = pl.multiple_of(bc * DENSE, 16)
        return [
            pltpu.async_copy(cx_hbm.at[pl.ds(bc, CHUNK)], cxv[b], sem_s[b]),
            pltpu.async_copy(cy_hbm.at[pl.ds(bc, CHUNK)], cyv[b], sem_s[b]),
            pltpu.async_copy(bxf_hbm.at[pl.ds(sb, CHUNK * DENSE)], bxv[b],
                             sem_s[b]),
            pltpu.async_copy(byf_hbm.at[pl.ds(sb, CHUNK * DENSE)], byv[b],
                             sem_s[b]),
        ]

    def addr_chunk(c, b):
        _, bc = bases(c)

        def addr_body(g, _):
            off = pl.multiple_of(g * 16, 16)
            cxg = cxv[b][pl.ds(off, 16)] + WD
            cyg = cyv[b][pl.ds(off, 16)] + WD
            pvec = bc + off + lax.iota(jnp.int32, 16)
            ch = (pvec.astype(jnp.float32) * jnp.float32(1.0 / PTS)
                  ).astype(jnp.int32)
            ac = ch * HW + cyg * W + cxg
            caddrv[b][pl.ds(off, 16)] = ac
            bidx = (off + lax.iota(jnp.int32, 16)) * DENSE
            for j in range(DENSE):
                bx = plsc.load_gather(bxv[b], [bidx + j]) - WD
                by = plsc.load_gather(byv[b], [bidx + j]) - 7
                addrv[b][j, pl.ds(off, 16)] = ac + by * W + bx
            return 0

        lax.fori_loop(0, GRP, addr_body, 0)

    def fire_gather(c, b):
        cps = [pltpu.async_copy(packed_hbm.at[addrv[b].at[j]],
                                datav[b].at[j], sem_g[b])
               for j in range(DENSE)]
        cps.append(pltpu.async_copy(packed_hbm.at[caddrv[b]],
                                    cdatav[b], sem_g[b]))
        return cps

    def stats_chunk(c, b):
        base, bc = bases(c)

        def stats_body(g, _):
            off = pl.multiple_of(g * 16, 16)
            cword = cdatav[b][pl.ds(off, 16)]
            onb = jnp.where((cword & (1 << 30)) != 0, 1.0, 0.0
                            ).astype(jnp.float32)
            pvec = bc + off + lax.iota(jnp.int32, 16)
            live = (pvec >= base) & (pvec < P0)
            onb = onb * jnp.where(live, 1.0, 0.0).astype(jnp.float32)

            posN = zero16
            sumP = zero16
            sumA = zero16
            for j in range(DENSE):
                w = datav[b][j, pl.ds(off, 16)]
                t = jnp.where(w < 0, 1.0, 0.0).astype(jnp.float32)
                dv = lax.bitcast_convert_type(w & jnp.int32(0x3FFFFFFF),
                                              jnp.float32)
                posN = posN + t
                sumP = sumP + dv * t
                sumA = sumA + dv
            negN = jnp.float32(DENSE) - posN
            posDen = jnp.maximum(posN, 1.0)
            negDen = jnp.maximum(negN, 1.0)
            posMean = sumP / posDen
            negMean = (sumA - sumP) / negDen
            vP = zero16
            vN = zero16
            for j in range(DENSE):
                w = datav[b][j, pl.ds(off, 16)]
                t = jnp.where(w < 0, 1.0, 0.0).astype(jnp.float32)
                dv = lax.bitcast_convert_type(w & jnp.int32(0x3FFFFFFF),
                                              jnp.float32)
                ep = dv - posMean
                en = dv - negMean
                vP = vP + ep * ep * t
                vN = vN + en * en * (1.0 - t)
            simP = _sqrt16(vP / posDen + 1e-14)
            simN = _sqrt16(vN / negDen + 1e-14)
            balance = jnp.where((posN > 4.5) & (negN > 4.5), 1.0, 0.0)
            sel = onb * balance.astype(jnp.float32)
            plsc.addupdate(accv.at[0], sel)
            plsc.addupdate(accv.at[1], simP * sel)
            plsc.addupdate(accv.at[2], simN * sel)
            plsc.addupdate(accv.at[3], (negMean - posMean) * sel)
            plsc.addupdate(accv.at[4], onb)
            return 0

        lax.fori_loop(0, GRP, stats_body, 0)

    st = fire_stage(0, 0)
    for cp in st:
        cp.wait()
    addr_chunk(0, 0)
    gcps = fire_gather(0, 0)
    st = fire_stage(1, 1)
    for c in range(1, NCHUNK):
        b, pb = c % 2, (c - 1) % 2
        nst = fire_stage(c + 1, pb) if c + 1 < NCHUNK else None
        for cp in st:
            cp.wait()
        addr_chunk(c, b)
        ngcps = fire_gather(c, b)
        for cp in gcps:
            cp.wait()
        stats_chunk(c - 1, pb)
        gcps = ngcps
        st = nst
    for cp in gcps:
        cp.wait()
    stats_chunk(NCHUNK - 1, (NCHUNK - 1) % 2)

    pltpu.sync_copy(accv, out_hbm.at[wid])


def _sc_sample(packed_flat, cxp, cyp, bxp, byp):
    mesh = plsc.VectorSubcoreMesh(core_axis_name="c", subcore_axis_name="s")
    return pl.kernel(
        _sc_body,
        out_type=jax.ShapeDtypeStruct((NWK, 5, 16), jnp.float32),
        mesh=mesh,
        compiler_params=pltpu.CompilerParams(needs_layout_passes=False),
        scratch_types=(
            [pltpu.VMEM((CHUNK,), jnp.int32)] * 4
            + [pltpu.VMEM((CHUNK * DENSE,), jnp.int32)] * 4
            + [pltpu.VMEM((DENSE, CHUNK), jnp.int32)] * 4
            + [pltpu.VMEM((CHUNK,), jnp.int32)] * 4
            + [pltpu.VMEM((5, 16), jnp.float32)]
            + [pltpu.SemaphoreType.DMA] * 4
        ),
    )(packed_flat, cxp, cyp, bxp, byp)


def kernel(disp, foregroundMask, centerx_raw, centery_raw, bx_raw, by_raw):
    mask3 = foregroundMask.reshape(BN, H, W)
    disp3 = disp.reshape(BN, H, W)
    packed = _pack_image(mask3, disp3).reshape(BN * HW)

    cxp = centerx_raw.astype(jnp.int32)
    cyp = centery_raw.astype(jnp.int32)
    bxp = bx_raw.astype(jnp.int32).reshape(-1)
    byp = by_raw.astype(jnp.int32).reshape(-1)

    parts = _sc_sample(packed, cxp, cyp, bxp, byp).sum(axis=(0, 2))
    count = parts[0]
    countSafe = jnp.where(count > 0, count, jnp.float32(1.0))
    lossSim = (parts[1] + parts[2]) / countSafe * jnp.float32(0.5)
    lossContrast = parts[3] / countSafe + jnp.float32(0.02)
    valid = (parts[4] >= 100) & (count >= 100)
    return (jnp.where(valid, lossSim, jnp.float32(0.0)),
            jnp.where(valid, lossContrast, jnp.float32(0.0)))


# speedup vs baseline: 9.7340x; 1.0498x over previous
import functools

import jax
import jax.numpy as jnp
from jax import lax
from jax.experimental import pallas as pl
from jax.experimental.pallas import tpu as pltpu
from jax.experimental.pallas import tpu_sc as plsc

WD = 11
PTS = 5000
DENSE = 20
BN = 10
H = 512
W = 1024
HW = H * W
P0 = PTS * BN

NC = 2
NS = 16
NWK = NC * NS
CHUNK = 128
NCHUNK = 13
PW = CHUNK * NCHUNK
PTOT = NWK * PW
GRP = CHUNK // 16


def _roll(x, shift, axis):
    return pltpu.roll(x, shift % x.shape[axis], axis)


def _pack_body(mask_ref, disp_ref, out_ref):
    m = mask_ref[0]
    d = disp_ref[0]
    cs = _roll(m, 1, 0) + 2.0 * m + _roll(m, -1, 0)
    g = jnp.abs(_roll(cs, -1, 1) - _roll(cs, 1, 1))
    rm = jnp.maximum(jnp.maximum(_roll(g, -1, 1), g), _roll(g, 1, 1))
    pm = jnp.maximum(jnp.maximum(_roll(rm, -1, 0), rm), _roll(rm, 1, 0))
    border = (pm > 3.1) & (d > 0.007)
    bits = lax.bitcast_convert_type(d, jnp.int32)
    bits = bits | jnp.where(m > 0.5, jnp.int32(-(2 ** 31)), jnp.int32(0))
    bits = bits | jnp.where(border, jnp.int32(1 << 30), jnp.int32(0))
    out_ref[...] = bits.reshape(H * W // 128, 128)


def _pack_image(mask3, disp3):
    return pl.pallas_call(
        _pack_body,
        grid=(BN,),
        in_specs=[pl.BlockSpec((1, H, W), lambda b: (b, 0, 0)),
                  pl.BlockSpec((1, H, W), lambda b: (b, 0, 0))],
        out_specs=pl.BlockSpec((H * W // 128, 128), lambda b: (b, 0)),
        out_shape=jax.ShapeDtypeStruct((BN * H * W // 128, 128), jnp.int32),
    )(mask3, disp3)


def _sqrt16(x):
    i = lax.bitcast_convert_type(x, jnp.int32)
    y = lax.bitcast_convert_type(jnp.int32(0x1FBD1DF5) + (i >> 1), jnp.float32)
    for _ in range(4):
        y = 0.5 * (y + x / y)
    return y


def _sc_body(packed_hbm, cx_hbm, cy_hbm, bxf_hbm, byf_hbm, out_hbm,
             cxv0, cxv1, cyv0, cyv1, bxv0, bxv1, byv0, byv1,
             addrv0, addrv1, datav0, datav1, caddrv0, caddrv1,
             cdatav0, cdatav1, accv, sem_s0, sem_s1, sem_g0, sem_g1):
    wid = lax.axis_index("s") * NC + lax.axis_index("c")
    zero16 = jnp.zeros((16,), jnp.float32)
    for i in range(5):
        accv[i] = zero16
    cxv, cyv, bxv, byv = (cxv0, cxv1), (cyv0, cyv1), (bxv0, bxv1), (byv0, byv1)
    addrv, datav = (addrv0, addrv1), (datav0, datav1)
    caddrv, cdatav = (caddrv0, caddrv1), (cdatav0, cdatav1)
    sem_s = (sem_s0, sem_s1)
    sem_g = (sem_g0, sem_g1)

    def bases(c):
        base = wid * PW + c * CHUNK
        bc = pl.multiple_of(jnp.minimum(base, P0 - CHUNK), 16)
        return base, bc

    def fire_stage(c, b):
        _, bc = bases(c)
        sb = pl.multiple_of(bc * DENSE, 16)
        return [
            pltpu.async_copy(cx_hbm.at[pl.ds(bc, CHUNK)], cxv[b], sem_s[b]),
            pltpu.async_copy(cy_hbm.at[pl.ds(bc, CHUNK)], cyv[b], sem_s[b]),
            pltpu.async_copy(bxf_hbm.at[pl.ds(sb, CHUNK * DENSE)], bxv[b],
                             sem_s[b]),
            pltpu.async_copy(byf_hbm.at[pl.ds(sb, CHUNK * DENSE)], byv[b],
                             sem_s[b]),
        ]

    def addr_chunk(c, b):
        _, bc = bases(c)

        def addr_body(g, _):
            off = pl.multiple_of(g * 16, 16)
            cxg = cxv[b][pl.ds(off, 16)] + WD
            cyg = cyv[b][pl.ds(off, 16)] + WD
            pvec = bc + off + lax.iota(jnp.int32, 16)
            ch = (pvec.astype(jnp.float32) * jnp.float32(1.0 / PTS)
                  ).astype(jnp.int32)
            ac = ch * HW + cyg * W + cxg
            caddrv[b][pl.ds(off, 16)] = ac
            bidx = (off + lax.iota(jnp.int32, 16)) * DENSE
            for j in range(DENSE):
                bx = plsc.load_gather(bxv[b], [bidx + j]) - WD
                by = plsc.load_gather(byv[b], [bidx + j]) - 7
                addrv[b][j, pl.ds(off, 16)] = ac + by * W + bx
            return 0

        lax.fori_loop(0, GRP, addr_body, 0)

    def fire_gather(c, b):
        cps = [pltpu.async_copy(packed_hbm.at[addrv[b].at[j]],
                                datav[b].at[j], sem_g[b])
               for j in range(DENSE)]
        cps.append(pltpu.async_copy(packed_hbm.at[caddrv[b]],
                                    cdatav[b], sem_g[b]))
        return cps

    def stats_chunk(c, b):
        base, bc = bases(c)

        def stats_body(g, _):
            off = pl.multiple_of(g * 16, 16)
            cword = cdatav[b][pl.ds(off, 16)]
            onb = jnp.where((cword & (1 << 30)) != 0, 1.0, 0.0
                            ).astype(jnp.float32)
            pvec = bc + off + lax.iota(jnp.int32, 16)
            live = (pvec >= base) & (pvec < P0)
            onb = onb * jnp.where(live, 1.0, 0.0).astype(jnp.float32)

            posN = zero16
            sumP = zero16
            sumA = zero16
            for j in range(DENSE):
                w = datav[b][j, pl.ds(off, 16)]
                t = jnp.where(w < 0, 1.0, 0.0).astype(jnp.float32)
                dv = lax.bitcast_convert_type(w & jnp.int32(0x3FFFFFFF),
                                              jnp.float32)
                posN = posN + t
                sumP = sumP + dv * t
                sumA = sumA + dv
            negN = jnp.float32(DENSE) - posN
            posDen = jnp.maximum(posN, 1.0)
            negDen = jnp.maximum(negN, 1.0)
            posMean = sumP / posDen
            negMean = (sumA - sumP) / negDen
            vP = zero16
            vN = zero16
            for j in range(DENSE):
                w = datav[b][j, pl.ds(off, 16)]
                t = jnp.where(w < 0, 1.0, 0.0).astype(jnp.float32)
                dv = lax.bitcast_convert_type(w & jnp.int32(0x3FFFFFFF),
                                              jnp.float32)
                ep = dv - posMean
                en = dv - negMean
                vP = vP + ep * ep * t
                vN = vN + en * en * (1.0 - t)
            simP = _sqrt16(vP / posDen + 1e-14)
            simN = _sqrt16(vN / negDen + 1e-14)
            balance = jnp.where((posN > 4.5) & (negN > 4.5), 1.0, 0.0)
            sel = onb * balance.astype(jnp.float32)
            plsc.addupdate(accv.at[0], sel)
            plsc.addupdate(accv.at[1], simP * sel)
            plsc.addupdate(accv.at[2], simN * sel)
            plsc.addupdate(accv.at[3], (negMean - posMean) * sel)
            plsc.addupdate(accv.at[4], onb)
            return 0

        lax.fori_loop(0, GRP, stats_body, 0)

    st = fire_stage(0, 0)
    for cp in st:
        cp.wait()
    addr_chunk(0, 0)
    gcps = fire_gather(0, 0)
    st = fire_stage(1, 1)
    for c in range(1, NCHUNK):
        b, pb = c % 2, (c - 1) % 2
        nst = fire_stage(c + 1, pb) if c + 1 < NCHUNK else None
        for cp in st:
            cp.wait()
        addr_chunk(c, b)
        ngcps = fire_gather(c, b)
        for cp in gcps:
            cp.wait()
        stats_chunk(c - 1, pb)
        gcps = ngcps
        st = nst
    for cp in gcps:
        cp.wait()
    stats_chunk(NCHUNK - 1, (NCHUNK - 1) % 2)

    pltpu.sync_copy(accv, out_hbm.at[wid])


def _sc_sample(packed_flat, cxp, cyp, bxp, byp):
    mesh = plsc.VectorSubcoreMesh(core_axis_name="c", subcore_axis_name="s")
    return pl.kernel(
        _sc_body,
        out_type=jax.ShapeDtypeStruct((NWK, 5, 16), jnp.float32),
        mesh=mesh,
        compiler_params=pltpu.CompilerParams(needs_layout_passes=False),
        scratch_types=(
            [pltpu.VMEM((CHUNK,), jnp.int32)] * 4
            + [pltpu.VMEM((CHUNK * DENSE,), jnp.int32)] * 4
            + [pltpu.VMEM((DENSE, CHUNK), jnp.int32)] * 4
            + [pltpu.VMEM((CHUNK,), jnp.int32)] * 4
            + [pltpu.VMEM((5, 16), jnp.float32)]
            + [pltpu.SemaphoreType.DMA] * 4
        ),
    )(packed_flat, cxp, cyp, bxp, byp)


def kernel(disp, foregroundMask, centerx_raw, centery_raw, bx_raw, by_raw):
    mask3 = foregroundMask.reshape(BN, H, W)
    disp3 = disp.reshape(BN, H, W)
    packed = _pack_image(mask3, disp3).reshape(BN * HW)

    cxp = centerx_raw.astype(jnp.int32)
    cyp = centery_raw.astype(jnp.int32)
    bxp = bx_raw.astype(jnp.int32).reshape(-1)
    byp = by_raw.astype(jnp.int32).reshape(-1)

    parts = _sc_sample(packed, cxp, cyp, bxp, byp).sum(axis=(0, 2))
    count = parts[0]
    countSafe = jnp.where(count > 0, count, jnp.float32(1.0))
    lossSim = (parts[1] + parts[2]) / countSafe * jnp.float32(0.5)
    lossContrast = parts[3] / countSafe + jnp.float32(0.02)
    valid = (parts[4] >= 100) & (count >= 100)
    return (jnp.where(valid, lossSim, jnp.float32(0.0)),
            jnp.where(valid, lossContrast, jnp.float32(0.0)))
